# Initial kernel scaffold; baseline (speedup 1.0000x reference)
import jax
import jax.numpy as jnp
from jax.experimental import pallas as pl


def kernel(enc1_w1, enc1_b1, enc1_w2, enc1_b2, enc2_w1, enc2_b1, enc2_w2, enc2_b2, enc3_w1, enc3_b1, enc3_w2, enc3_b2, bn_w1, bn_b1, bn_w2, bn_b2, up3_w, up3_b, up2_w, up2_b, up1_w, up1_b, out_w, out_b, x):
    raise NotImplementedError("write your pallas kernel here")


# speedup vs baseline: 3.6106x; 3.6106x over previous
import numpy as np
import jax
import jax.numpy as jnp
from jax.experimental import pallas as pl
from jax.experimental.pallas import tpu as pltpu

_PAD = 64
_B = 8
_TAPS9 = [(dh, dw) for dh in (-1, 0, 1) for dw in (-1, 0, 1)]
_BF = jnp.bfloat16


def _tap_masks_np(S, B):
    P = S * S
    m = np.zeros((9, 1, P), np.float32)
    for t, (dh, dw) in enumerate(_TAPS9):
        for h in range(S):
            for w in range(S):
                if 0 <= h + dh < S and 0 <= w + dw < S:
                    m[t, 0, h * S + w] = 1.0
    return np.tile(m, (1, 1, B))


def _pool_select_np(S, B):
    So = S // 2
    g = np.zeros((S * S, So * So), np.float32)
    for ho in range(So):
        for wo in range(So):
            g[(2 * ho) * S + 2 * wo, ho * So + wo] = 1.0
    return np.kron(np.eye(B, dtype=np.float32), g)


def _upsample_scatter_np(S, B):
    p = np.zeros((4, S * S, 4 * S * S), np.float32)
    for kh in range(2):
        for kw in range(2):
            t = kh * 2 + kw
            for h in range(S):
                for w in range(S):
                    p[t, h * S + w, (2 * h + kh) * (2 * S) + (2 * w + kw)] = 1.0
    return np.concatenate(
        [np.kron(np.eye(B, dtype=np.float32), p[t]) for t in range(4)], axis=0)


def _stage(pad, val):
    cin, L = val.shape
    pad[:cin, pl.ds(_PAD, L)] = val.astype(_BF)


def _conv3x3_relu(pad, cin, S, L, w_ref, b_ref, m_ref):
    taps = []
    for t, (dh, dw) in enumerate(_TAPS9):
        off = dh * S + dw
        taps.append(pad[:cin, pl.ds(_PAD + off, L)] * m_ref[t])
    big = jnp.concatenate(taps, axis=0)
    acc = jnp.dot(w_ref[...], big, preferred_element_type=jnp.float32)
    return jnp.maximum(acc + b_ref[...], 0.0)


def _maxpool2x2(pad, cin, S, L, g_ref):
    x0 = pad[:cin, pl.ds(_PAD, L)]
    t1 = pad[:cin, pl.ds(_PAD + 1, L)]
    t2 = pad[:cin, pl.ds(_PAD + S, L)]
    t3 = pad[:cin, pl.ds(_PAD + S + 1, L)]
    m = jnp.maximum(jnp.maximum(x0, t1), jnp.maximum(t2, t3))
    return jnp.dot(m, g_ref[...], preferred_element_type=jnp.float32)


def _conv_transpose2x2(xcat, w_ref, b_ref, p_ref):
    parts = [jnp.dot(w_ref[t], xcat, preferred_element_type=jnp.float32)
             for t in range(4)]
    alane = jnp.concatenate(parts, axis=1).astype(_BF)
    return jnp.dot(alane, p_ref[...], preferred_element_type=jnp.float32) + b_ref[...]


def _unet_kernel(x_ref, m16, m8, m4, m2,
                 w11, b11, w12, b12, w21, b21, w22, b22,
                 w31, b31, w32, b32, wb1, bb1, wb2, bb2,
                 g1, g2, g3, u3w, u3b, u2w, u2b, u1w, u1b,
                 p2, p4, p8, ow, ob, o_ref, pad):
    pad[...] = jnp.zeros_like(pad)
    C, P = x_ref.shape[1], x_ref.shape[2]
    c2, c4, c8 = C // 2, C // 4, C // 8
    L1, L2, L3, L4 = _B * P, _B * P // 4, _B * P // 16, _B * P // 64

    for b in range(_B):
        pad[:C, pl.ds(_PAD + b * P, P)] = x_ref[b].astype(_BF)

    t = _conv3x3_relu(pad, C, 16, L1, w11, b11, m16)
    _stage(pad, t)
    e1 = _conv3x3_relu(pad, c2, 16, L1, w12, b12, m16)
    e1b = e1.astype(_BF)
    _stage(pad, e1b)
    p1 = _maxpool2x2(pad, c2, 16, L1, g1)
    _stage(pad, p1)
    t = _conv3x3_relu(pad, c2, 8, L2, w21, b21, m8)
    _stage(pad, t)
    e2 = _conv3x3_relu(pad, c4, 8, L2, w22, b22, m8)
    e2b = e2.astype(_BF)
    _stage(pad, e2b)
    p2v = _maxpool2x2(pad, c4, 8, L2, g2)
    _stage(pad, p2v)
    t = _conv3x3_relu(pad, c4, 4, L3, w31, b31, m4)
    _stage(pad, t)
    e3 = _conv3x3_relu(pad, c8, 4, L3, w32, b32, m4)
    e3b = e3.astype(_BF)
    _stage(pad, e3b)
    p3v = _maxpool2x2(pad, c8, 4, L3, g3)
    _stage(pad, p3v)
    t = _conv3x3_relu(pad, c8, 2, L4, wb1, bb1, m2)
    _stage(pad, t)
    bn = _conv3x3_relu(pad, c8, 2, L4, wb2, bb2, m2)

    u3 = _conv_transpose2x2(bn.astype(_BF), u3w, u3b, p2)
    u2 = _conv_transpose2x2(
        jnp.concatenate([u3.astype(_BF), e3b], axis=0), u2w, u2b, p4)
    u1 = _conv_transpose2x2(
        jnp.concatenate([u2.astype(_BF), e2b], axis=0), u1w, u1b, p8)
    fin = jnp.concatenate([u1.astype(_BF), e1b], axis=0)
    out = jnp.dot(ow[...], fin, preferred_element_type=jnp.float32) + ob[...]
    for b in range(_B):
        o_ref[b] = out[:, b * P:(b + 1) * P]


def _flat9(w):
    return jnp.transpose(w, (3, 0, 1, 2)).reshape(w.shape[3], -1).astype(_BF)


def _t4(w):
    return jnp.transpose(w, (0, 1, 3, 2)).reshape(4, w.shape[3], w.shape[2]).astype(_BF)


def _col(b):
    return b.reshape(-1, 1)


def kernel(enc1_w1, enc1_b1, enc1_w2, enc1_b2, enc2_w1, enc2_b1, enc2_w2,
           enc2_b2, enc3_w1, enc3_b1, enc3_w2, enc3_b2, bn_w1, bn_b1, bn_w2,
           bn_b2, up3_w, up3_b, up2_w, up2_b, up1_w, up1_b, out_w, out_b, x):
    N, C, H, W = x.shape
    P = H * W
    B = _B
    bf = lambda a: jnp.asarray(a, dtype=_BF)

    consts = (
        bf(_tap_masks_np(16, B)), bf(_tap_masks_np(8, B)),
        bf(_tap_masks_np(4, B)), bf(_tap_masks_np(2, B)),
        _flat9(enc1_w1), _col(enc1_b1), _flat9(enc1_w2), _col(enc1_b2),
        _flat9(enc2_w1), _col(enc2_b1), _flat9(enc2_w2), _col(enc2_b2),
        _flat9(enc3_w1), _col(enc3_b1), _flat9(enc3_w2), _col(enc3_b2),
        _flat9(bn_w1), _col(bn_b1), _flat9(bn_w2), _col(bn_b2),
        bf(_pool_select_np(16, B)), bf(_pool_select_np(8, B)),
        bf(_pool_select_np(4, B)),
        _t4(up3_w), _col(up3_b), _t4(up2_w), _col(up2_b), _t4(up1_w), _col(up1_b),
        bf(_upsample_scatter_np(2, B)), bf(_upsample_scatter_np(4, B)),
        bf(_upsample_scatter_np(8, B)),
        out_w.T.astype(_BF), _col(out_b),
    )

    x2 = x.reshape(N, C, P)
    in_specs = [pl.BlockSpec((B, C, P), lambda n: (n, 0, 0))]
    for a in consts:
        in_specs.append(pl.BlockSpec(a.shape, lambda n, _nd=a.ndim: (0,) * _nd))

    pad_lanes = (_PAD + B * P + _PAD + 127) // 128 * 128

    out = pl.pallas_call(
        _unet_kernel,
        out_shape=jax.ShapeDtypeStruct((N, C, P), jnp.float32),
        grid=(N // B,),
        in_specs=in_specs,
        out_specs=pl.BlockSpec((B, C, P), lambda n: (n, 0, 0)),
        scratch_shapes=[pltpu.VMEM((C, pad_lanes), _BF)],
        compiler_params=pltpu.CompilerParams(
            dimension_semantics=("parallel",),
            vmem_limit_bytes=64 * 1024 * 1024),
    )(x2, *consts)
    return out.reshape(N, C, H, W)


# speedup vs baseline: 4.5572x; 1.2622x over previous
import numpy as np
import jax
import jax.numpy as jnp
from jax.experimental import pallas as pl
from jax.experimental.pallas import tpu as pltpu

_PAD = 64
_B = 8
_TAPS9 = [(dh, dw) for dh in (-1, 0, 1) for dw in (-1, 0, 1)]
_BF = jnp.bfloat16


def _tap_masks_np(S, B):
    P = S * S
    m = np.zeros((9, 1, P), np.float32)
    for t, (dh, dw) in enumerate(_TAPS9):
        for h in range(S):
            for w in range(S):
                if 0 <= h + dh < S and 0 <= w + dw < S:
                    m[t, 0, h * S + w] = 1.0
    return np.tile(m, (1, 1, B))


def _pool_select_np(S, B):
    So = S // 2
    g = np.zeros((S * S, So * So), np.float32)
    for ho in range(So):
        for wo in range(So):
            g[(2 * ho) * S + 2 * wo, ho * So + wo] = 1.0
    return np.kron(np.eye(B, dtype=np.float32), g)


def _upsample_scatter_np(S, B):
    p = np.zeros((4, S * S, 4 * S * S), np.float32)
    for kh in range(2):
        for kw in range(2):
            t = kh * 2 + kw
            for h in range(S):
                for w in range(S):
                    p[t, h * S + w, (2 * h + kh) * (2 * S) + (2 * w + kw)] = 1.0
    return np.concatenate(
        [np.kron(np.eye(B, dtype=np.float32), p[t]) for t in range(4)], axis=0)


def _stage(pad, val):
    cin, L = val.shape
    pad[:cin, pl.ds(_PAD, L)] = val.astype(_BF)


def _conv3x3_relu(pad, cin, S, L, w_ref, b_ref, m_ref):
    taps = []
    for t, (dh, dw) in enumerate(_TAPS9):
        off = dh * S + dw
        taps.append(pad[:cin, pl.ds(_PAD + off, L)] * m_ref[t])
    big = jnp.concatenate(taps, axis=0)
    acc = jnp.dot(w_ref[...], big, preferred_element_type=jnp.float32)
    return jnp.maximum(acc + b_ref[...], 0.0)


def _maxpool2x2(pad, cin, S, L, g_ref):
    x0 = pad[:cin, pl.ds(_PAD, L)]
    t1 = pad[:cin, pl.ds(_PAD + 1, L)]
    t2 = pad[:cin, pl.ds(_PAD + S, L)]
    t3 = pad[:cin, pl.ds(_PAD + S + 1, L)]
    m = jnp.maximum(jnp.maximum(x0, t1), jnp.maximum(t2, t3))
    return jnp.dot(m, g_ref[...], preferred_element_type=jnp.float32)


def _conv_transpose2x2(xcat, w_ref, b_ref, p_ref):
    parts = [jnp.dot(w_ref[t], xcat, preferred_element_type=jnp.float32)
             for t in range(4)]
    alane = jnp.concatenate(parts, axis=1).astype(_BF)
    return jnp.dot(alane, p_ref[...], preferred_element_type=jnp.float32) + b_ref[...]


def _unet_kernel(x_ref, m16, m8, m4, m2,
                 w11, b11, w12, b12, w21, b21, w22, b22,
                 w31, b31, w32, b32, wb1, bb1, wb2, bb2,
                 g1, g2, g3, u3w, u3b, u2w, u2b, u1w, u1b,
                 p2, p4, p8, ow, ob, o_ref, pad_a, pad_b):
    pads = (pad_a, pad_b)
    CH = len(pads)
    C, P = x_ref.shape[1], x_ref.shape[2]
    c2, c4, c8 = C // 2, C // 4, C // 8
    L1, L2, L3, L4 = _B * P, _B * P // 4, _B * P // 16, _B * P // 64

    def both(f):
        return [f(i) for i in range(CH)]

    def stage_all(vals):
        for i in range(CH):
            _stage(pads[i], vals[i])

    for i in range(CH):
        pads[i][...] = jnp.zeros_like(pads[i])
    for i in range(CH):
        for b in range(_B):
            pads[i][:C, pl.ds(_PAD + b * P, P)] = x_ref[i * _B + b].astype(_BF)

    stage_all(both(lambda i: _conv3x3_relu(pads[i], C, 16, L1, w11, b11, m16)))
    e1b = [v.astype(_BF)
           for v in both(lambda i: _conv3x3_relu(pads[i], c2, 16, L1, w12, b12, m16))]
    stage_all(e1b)
    stage_all(both(lambda i: _maxpool2x2(pads[i], c2, 16, L1, g1)))
    stage_all(both(lambda i: _conv3x3_relu(pads[i], c2, 8, L2, w21, b21, m8)))
    e2b = [v.astype(_BF)
           for v in both(lambda i: _conv3x3_relu(pads[i], c4, 8, L2, w22, b22, m8))]
    stage_all(e2b)
    stage_all(both(lambda i: _maxpool2x2(pads[i], c4, 8, L2, g2)))
    stage_all(both(lambda i: _conv3x3_relu(pads[i], c4, 4, L3, w31, b31, m4)))
    e3b = [v.astype(_BF)
           for v in both(lambda i: _conv3x3_relu(pads[i], c8, 4, L3, w32, b32, m4))]
    stage_all(e3b)
    stage_all(both(lambda i: _maxpool2x2(pads[i], c8, 4, L3, g3)))
    stage_all(both(lambda i: _conv3x3_relu(pads[i], c8, 2, L4, wb1, bb1, m2)))
    bn = both(lambda i: _conv3x3_relu(pads[i], c8, 2, L4, wb2, bb2, m2))

    u3 = both(lambda i: _conv_transpose2x2(bn[i].astype(_BF), u3w, u3b, p2))
    u2 = both(lambda i: _conv_transpose2x2(
        jnp.concatenate([u3[i].astype(_BF), e3b[i]], axis=0), u2w, u2b, p4))
    u1 = both(lambda i: _conv_transpose2x2(
        jnp.concatenate([u2[i].astype(_BF), e2b[i]], axis=0), u1w, u1b, p8))
    out = both(lambda i: jnp.dot(
        ow[...], jnp.concatenate([u1[i].astype(_BF), e1b[i]], axis=0),
        preferred_element_type=jnp.float32) + ob[...])
    for i in range(CH):
        for b in range(_B):
            o_ref[i * _B + b] = out[i][:, b * P:(b + 1) * P]


def _flat9(w):
    return jnp.transpose(w, (3, 0, 1, 2)).reshape(w.shape[3], -1).astype(_BF)


def _t4(w):
    return jnp.transpose(w, (0, 1, 3, 2)).reshape(4, w.shape[3], w.shape[2]).astype(_BF)


def _col(b):
    return b.reshape(-1, 1)


def kernel(enc1_w1, enc1_b1, enc1_w2, enc1_b2, enc2_w1, enc2_b1, enc2_w2,
           enc2_b2, enc3_w1, enc3_b1, enc3_w2, enc3_b2, bn_w1, bn_b1, bn_w2,
           bn_b2, up3_w, up3_b, up2_w, up2_b, up1_w, up1_b, out_w, out_b, x):
    N, C, H, W = x.shape
    P = H * W
    B = _B
    bf = lambda a: jnp.asarray(a, dtype=_BF)

    consts = (
        bf(_tap_masks_np(16, B)), bf(_tap_masks_np(8, B)),
        bf(_tap_masks_np(4, B)), bf(_tap_masks_np(2, B)),
        _flat9(enc1_w1), _col(enc1_b1), _flat9(enc1_w2), _col(enc1_b2),
        _flat9(enc2_w1), _col(enc2_b1), _flat9(enc2_w2), _col(enc2_b2),
        _flat9(enc3_w1), _col(enc3_b1), _flat9(enc3_w2), _col(enc3_b2),
        _flat9(bn_w1), _col(bn_b1), _flat9(bn_w2), _col(bn_b2),
        bf(_pool_select_np(16, B)), bf(_pool_select_np(8, B)),
        bf(_pool_select_np(4, B)),
        _t4(up3_w), _col(up3_b), _t4(up2_w), _col(up2_b), _t4(up1_w), _col(up1_b),
        bf(_upsample_scatter_np(2, B)), bf(_upsample_scatter_np(4, B)),
        bf(_upsample_scatter_np(8, B)),
        out_w.T.astype(_BF), _col(out_b),
    )

    x2 = x.reshape(N, C, P)
    G = 2 * B
    in_specs = [pl.BlockSpec((G, C, P), lambda n: (n, 0, 0))]
    for a in consts:
        in_specs.append(pl.BlockSpec(a.shape, lambda n, _nd=a.ndim: (0,) * _nd))

    pad_lanes = (_PAD + B * P + _PAD + 127) // 128 * 128

    out = pl.pallas_call(
        _unet_kernel,
        out_shape=jax.ShapeDtypeStruct((N, C, P), jnp.float32),
        grid=(N // G,),
        in_specs=in_specs,
        out_specs=pl.BlockSpec((G, C, P), lambda n: (n, 0, 0)),
        scratch_shapes=[pltpu.VMEM((C, pad_lanes), _BF),
                        pltpu.VMEM((C, pad_lanes), _BF)],
        compiler_params=pltpu.CompilerParams(
            dimension_semantics=("parallel",),
            vmem_limit_bytes=64 * 1024 * 1024),
    )(x2, *consts)
    return out.reshape(N, C, H, W)
